# Initial kernel scaffold; baseline (speedup 1.0000x reference)
import jax
import jax.numpy as jnp
from jax.experimental import pallas as pl


def kernel(aa_idx, physchem, aa_table, W_phys, b_phys, pos_table, gamma, beta):
    raise NotImplementedError("write your pallas kernel here")


# speedup vs baseline: 1.8277x; 1.8277x over previous
import functools

import jax
import jax.numpy as jnp
from jax.experimental import pallas as pl
from jax.experimental.pallas import tpu as pltpu

N_AA = 21
D = 64
LSEQ = 512
RB = 4096
NFEAT = 32


def _emb_kernel(idx_ref, phys_ref, tab_ref, bp_ref, pos_ref, g_ref, b_ref,
                out_ref):
    idx = idx_ref[...]
    phys = phys_ref[...]
    lane = jax.lax.broadcasted_iota(jnp.int32, (RB, NFEAT), 1)
    f = (lane == idx).astype(jnp.float32)
    for j in range(3):
        f = jnp.where(lane == N_AA + j, phys[:, j:j + 1], f)
    emb = jnp.dot(f, tab_ref[...], preferred_element_type=jnp.float32)
    emb = emb + bp_ref[...]
    emb = (emb.reshape(RB // LSEQ, LSEQ, D) + pos_ref[...][None, :, :])
    emb = emb.reshape(RB, D)
    mean = jnp.mean(emb, axis=-1, keepdims=True)
    var = jnp.mean(emb * emb, axis=-1, keepdims=True) - mean * mean
    inv = jax.lax.rsqrt(var + 1e-5)
    out_ref[...] = (emb - mean) * inv * g_ref[...] + b_ref[...]


@jax.jit
def kernel(aa_idx, physchem, aa_table, W_phys, b_phys, pos_table, gamma, beta):
    Bsz, Ls = aa_idx.shape
    n_rows = Bsz * Ls
    idx = aa_idx.reshape(n_rows, 1).astype(jnp.int32)
    phys = physchem.reshape(n_rows, 3)
    table = (jnp.zeros((NFEAT, D), jnp.float32)
             .at[:N_AA].set(aa_table)
             .at[N_AA:N_AA + 3].set(W_phys))
    bp = b_phys.reshape(1, D)
    g = gamma.reshape(1, D)
    b = beta.reshape(1, D)

    grid = (n_rows // RB,)
    out = pl.pallas_call(
        _emb_kernel,
        grid=grid,
        in_specs=[
            pl.BlockSpec((RB, 1), lambda i: (i, 0)),
            pl.BlockSpec((RB, 3), lambda i: (i, 0)),
            pl.BlockSpec((NFEAT, D), lambda i: (0, 0)),
            pl.BlockSpec((1, D), lambda i: (0, 0)),
            pl.BlockSpec((LSEQ, D), lambda i: (0, 0)),
            pl.BlockSpec((1, D), lambda i: (0, 0)),
            pl.BlockSpec((1, D), lambda i: (0, 0)),
        ],
        out_specs=pl.BlockSpec((RB, D), lambda i: (i, 0)),
        out_shape=jax.ShapeDtypeStruct((n_rows, D), jnp.float32),
        compiler_params=pltpu.CompilerParams(
            dimension_semantics=("arbitrary",),
        ),
    )(idx, phys, table, bp, pos_table, g, b)
    return out.reshape(Bsz, Ls, D)


# speedup vs baseline: 21.1496x; 11.5716x over previous
import jax
import jax.numpy as jnp
from jax.experimental import pallas as pl
from jax.experimental.pallas import tpu as pltpu

N_AA = 21
D = 64
LSEQ = 512
KOH = 24
KF = KOH + 3
G = 64


def _emb_kernel(idx_ref, phys_ref, TX_ref, pos_ref, pm_ref, g_ref, b_ref,
                out_ref):
    TX = TX_ref[...]
    posb = pos_ref[...]
    pmean = pm_ref[...]
    gT = g_ref[...]
    bT = b_ref[...]
    dn = (((0,), (0,)), ((), ()))
    for g in range(G):
        idx = idx_ref[g:g + 1, :]
        s = jax.lax.broadcasted_iota(jnp.int32, (KOH, LSEQ), 0)
        oh = (s == idx).astype(jnp.float32)
        feat = jnp.concatenate([oh, phys_ref[g]], axis=0)
        ext = jax.lax.dot_general(TX, feat, dn,
                                  preferred_element_type=jnp.float32)
        emb = ext[:D] + posb
        mean = ext[D:D + 1] + pmean
        msq = jnp.mean(emb * emb, axis=0, keepdims=True)
        var = msq - mean * mean
        inv = jax.lax.rsqrt(var + 1e-5)
        out_ref[g] = (emb - mean) * inv * gT + bT


@jax.jit
def kernel(aa_idx, physchem, aa_table, W_phys, b_phys, pos_table, gamma, beta):
    Bsz, Ls = aa_idx.shape
    T27 = (jnp.zeros((KF, D), jnp.float32)
           .at[:N_AA].set(aa_table)
           .at[KOH:].set(W_phys))
    TX = (jnp.zeros((KF, D + 8), jnp.float32)
          .at[:, :D].set(T27)
          .at[:, D].set(jnp.sum(T27, axis=1) / D))
    posb = pos_table.T + b_phys[:, None]
    pmean = jnp.mean(posb, axis=0, keepdims=True)
    gT = gamma.reshape(D, 1)
    bT = beta.reshape(D, 1)

    out = pl.pallas_call(
        _emb_kernel,
        grid=(Bsz // G,),
        in_specs=[
            pl.BlockSpec((G, LSEQ), lambda i: (i, 0)),
            pl.BlockSpec((G, 3, LSEQ), lambda i: (i, 0, 0)),
            pl.BlockSpec((KF, D + 8), lambda i: (0, 0)),
            pl.BlockSpec((D, LSEQ), lambda i: (0, 0)),
            pl.BlockSpec((1, LSEQ), lambda i: (0, 0)),
            pl.BlockSpec((D, 1), lambda i: (0, 0)),
            pl.BlockSpec((D, 1), lambda i: (0, 0)),
        ],
        out_specs=pl.BlockSpec((G, D, LSEQ), lambda i: (i, 0, 0)),
        out_shape=jax.ShapeDtypeStruct((Bsz, D, LSEQ), jnp.float32),
        compiler_params=pltpu.CompilerParams(
            dimension_semantics=("arbitrary",),
        ),
    )(aa_idx.astype(jnp.int32), physchem.transpose(0, 2, 1), TX,
      posb, pmean, gT, bT)
    return out.transpose(0, 2, 1)


# speedup vs baseline: 22.0405x; 1.0421x over previous
import jax
import jax.numpy as jnp
from jax.experimental import pallas as pl
from jax.experimental.pallas import tpu as pltpu

N_AA = 21
D = 64
LSEQ = 512
KOH = 24
G = 64


def _emb_kernel(idx_ref, phys_ref, T24_ref, W_ref, pos_ref, g_ref, b_ref,
                out_ref):
    T24 = T24_ref[...]
    W = W_ref[...]
    posb = pos_ref[...]
    gT = g_ref[...]
    bT = b_ref[...]
    dn = (((0,), (0,)), ((), ()))
    for g in range(G):
        idx = idx_ref[g:g + 1, :]
        s = jax.lax.broadcasted_iota(jnp.int32, (KOH, LSEQ), 0)
        oh = (s == idx).astype(jnp.float32)
        emb = jax.lax.dot_general(T24, oh, dn,
                                  preferred_element_type=jnp.float32)
        emb = emb + jax.lax.dot_general(W, phys_ref[g], dn,
                                        preferred_element_type=jnp.float32)
        emb = emb + posb
        mean = jnp.mean(emb, axis=0, keepdims=True)
        msq = jnp.mean(emb * emb, axis=0, keepdims=True)
        var = msq - mean * mean
        inv = jax.lax.rsqrt(var + 1e-5)
        out_ref[g] = (emb - mean) * inv * gT + bT


@jax.jit
def kernel(aa_idx, physchem, aa_table, W_phys, b_phys, pos_table, gamma, beta):
    Bsz, Ls = aa_idx.shape
    T24 = jnp.zeros((KOH, D), jnp.float32).at[:N_AA].set(aa_table)
    posb = pos_table.T + b_phys[:, None]
    gT = gamma.reshape(D, 1)
    bT = beta.reshape(D, 1)

    out = pl.pallas_call(
        _emb_kernel,
        grid=(Bsz // G,),
        in_specs=[
            pl.BlockSpec((G, LSEQ), lambda i: (i, 0)),
            pl.BlockSpec((G, 3, LSEQ), lambda i: (i, 0, 0)),
            pl.BlockSpec((KOH, D), lambda i: (0, 0)),
            pl.BlockSpec((3, D), lambda i: (0, 0)),
            pl.BlockSpec((D, LSEQ), lambda i: (0, 0)),
            pl.BlockSpec((D, 1), lambda i: (0, 0)),
            pl.BlockSpec((D, 1), lambda i: (0, 0)),
        ],
        out_specs=pl.BlockSpec((G, D, LSEQ), lambda i: (i, 0, 0)),
        out_shape=jax.ShapeDtypeStruct((Bsz, D, LSEQ), jnp.float32),
        compiler_params=pltpu.CompilerParams(
            dimension_semantics=("arbitrary",),
        ),
    )(aa_idx.astype(jnp.int32), physchem.transpose(0, 2, 1), T24, W_phys,
      posb, gT, bT)
    return out.transpose(0, 2, 1)


# speedup vs baseline: 22.9888x; 1.0430x over previous
import jax
import jax.numpy as jnp
from jax.experimental import pallas as pl
from jax.experimental.pallas import tpu as pltpu

N_AA = 21
D = 64
LSEQ = 512
KOH = 24
G = 64


def _emb_kernel(idx_ref, phys_ref, T24_ref, W_ref, pos_ref, pm_ref, g_ref,
                b_ref, out_ref):
    T24 = T24_ref[...]
    W = W_ref[...]
    posb = pos_ref[...]
    pmean = pm_ref[...]
    gT = g_ref[...]
    bT = b_ref[...]
    dn = (((0,), (0,)), ((), ()))
    for g in range(G):
        idx = idx_ref[g:g + 1, :]
        s = jax.lax.broadcasted_iota(jnp.int32, (KOH, LSEQ), 0)
        oh = (s == idx).astype(jnp.float32)
        ext = jax.lax.dot_general(T24, oh, dn,
                                  preferred_element_type=jnp.float32)
        ext = ext + jax.lax.dot_general(W, phys_ref[g], dn,
                                        preferred_element_type=jnp.float32)
        emb = ext[:D] + posb
        mean = ext[D:D + 1] + pmean
        msq = jnp.mean(emb * emb, axis=0, keepdims=True)
        var = msq - mean * mean
        inv = jax.lax.rsqrt(var + 1e-5)
        out_ref[g] = (emb - mean) * inv * gT + bT


@jax.jit
def kernel(aa_idx, physchem, aa_table, W_phys, b_phys, pos_table, gamma, beta):
    Bsz, Ls = aa_idx.shape
    T24 = (jnp.zeros((KOH, D + 8), jnp.float32)
           .at[:N_AA, :D].set(aa_table)
           .at[:N_AA, D].set(jnp.sum(aa_table, axis=1) / D))
    Wx = (jnp.zeros((3, D + 8), jnp.float32)
          .at[:, :D].set(W_phys)
          .at[:, D].set(jnp.sum(W_phys, axis=1) / D))
    posb = pos_table.T + b_phys[:, None]
    pmean = jnp.mean(posb, axis=0, keepdims=True)
    gT = gamma.reshape(D, 1)
    bT = beta.reshape(D, 1)

    out = pl.pallas_call(
        _emb_kernel,
        grid=(Bsz // G,),
        in_specs=[
            pl.BlockSpec((G, LSEQ), lambda i: (i, 0)),
            pl.BlockSpec((G, 3, LSEQ), lambda i: (i, 0, 0)),
            pl.BlockSpec((KOH, D + 8), lambda i: (0, 0)),
            pl.BlockSpec((3, D + 8), lambda i: (0, 0)),
            pl.BlockSpec((D, LSEQ), lambda i: (0, 0)),
            pl.BlockSpec((1, LSEQ), lambda i: (0, 0)),
            pl.BlockSpec((D, 1), lambda i: (0, 0)),
            pl.BlockSpec((D, 1), lambda i: (0, 0)),
        ],
        out_specs=pl.BlockSpec((G, D, LSEQ), lambda i: (i, 0, 0)),
        out_shape=jax.ShapeDtypeStruct((Bsz, D, LSEQ), jnp.float32),
        compiler_params=pltpu.CompilerParams(
            dimension_semantics=("arbitrary",),
        ),
    )(aa_idx.astype(jnp.int32), physchem.transpose(0, 2, 1), T24, Wx,
      posb, pmean, gT, bT)
    return out.transpose(0, 2, 1)
